# baseline (speedup 1.0000x reference)
import jax
import jax.numpy as jnp
from jax import lax
from jax.experimental import pallas as pl
from jax.experimental.pallas import tpu as pltpu
from jax.experimental.pallas import tpu_sc as plsc

B, L, D = 1024, 200, 128
N = B * L
NC, NS = 2, 16
NW = NC * NS
PER_W = N // NW
CH = 128
SCH = 256
GPC = SCH // CH
NCHUNK = PER_W // SCH
NBUF = 3


def _is_type_b(c):
    return c % 5 < 2


def _pe_add_body(x_hbm, idx_hbm, pe_hbm, ar_hbm, out_hbm,
                 idx_v, iota_v, xb_v, sem_x, sem_g, sem_o):
    wid = lax.axis_index("s") * NC + lax.axis_index("c")
    base = wid * PER_W

    pltpu.sync_copy(idx_hbm.at[pl.ds(base, PER_W)], idx_v)
    pltpu.sync_copy(ar_hbm.at[pl.ds(base, PER_W)], iota_v)

    def off(c):
        return base + c * SCH

    def _indirect(c, b, table, ind, sem, add):
        return [
            (table.at[ind.at[pl.ds(c * SCH + g * CH, CH)]],
             xb_v.at[b, pl.ds(g * CH, CH)], sem.at[b], add)
            for g in range(GPC)
        ]

    def s1_args(c, b):
        if _is_type_b(c):
            return _indirect(c, b, pe_hbm, idx_v, sem_x, False)
        return [(x_hbm.at[pl.ds(off(c), SCH)], xb_v.at[b], sem_x.at[b], False)]

    def s2_args(c, b):
        if _is_type_b(c):
            return _indirect(c, b, x_hbm, iota_v, sem_g, True)
        return _indirect(c, b, pe_hbm, idx_v, sem_g, True)

    def fire(args):
        for src, dst, sem, add in args:
            pltpu.async_copy(src, dst, sem, add=add)

    def wait(args):
        for src, dst, sem, add in args:
            pltpu.make_async_copy(src, dst, sem).wait()

    def fire_out(c, b):
        pltpu.async_copy(xb_v.at[b], out_hbm.at[pl.ds(off(c), SCH)],
                         sem_o.at[b])

    def wait_out(c, b):
        pltpu.make_async_copy(
            xb_v.at[b], out_hbm.at[pl.ds(off(c), SCH)], sem_o.at[b]).wait()

    for t in range(NCHUNK + 2):
        if t >= 2:
            c = t - 2
            wait(s2_args(c, c % NBUF))
            fire_out(c, c % NBUF)
        if 1 <= t <= NCHUNK:
            c = t - 1
            wait(s1_args(c, c % NBUF))
            fire(s2_args(c, c % NBUF))
        if t < NCHUNK:
            if t >= NBUF:
                wait_out(t - NBUF, t % NBUF)
            fire(s1_args(t, t % NBUF))
    for k in range(NBUF):
        c = NCHUNK - NBUF + k
        wait_out(c, c % NBUF)


@jax.jit
def kernel(x, tss_indexes, pe):
    xf = x.reshape(N, D)
    idx = tss_indexes.reshape(N).astype(jnp.int32)
    mesh = plsc.VectorSubcoreMesh(
        core_axis_name="c", subcore_axis_name="s",
        num_cores=NC, num_subcores=NS,
    )
    out = pl.kernel(
        _pe_add_body,
        out_type=jax.ShapeDtypeStruct((N, D), jnp.float32),
        mesh=mesh,
        scratch_types=[
            pltpu.VMEM((PER_W,), jnp.int32),
            pltpu.VMEM((PER_W,), jnp.int32),
            pltpu.VMEM((NBUF, SCH, D), jnp.float32),
            pltpu.SemaphoreType.DMA((NBUF,)),
            pltpu.SemaphoreType.DMA((NBUF,)),
            pltpu.SemaphoreType.DMA((NBUF,)),
        ],
    )(xf, idx, pe, jnp.arange(N, dtype=jnp.int32))
    return out.reshape(B, L, D)


# speedup vs baseline: 1.0063x; 1.0063x over previous
import jax
import jax.numpy as jnp
from jax import lax
from jax.experimental import pallas as pl
from jax.experimental.pallas import tpu as pltpu
from jax.experimental.pallas import tpu_sc as plsc

B, L, D = 1024, 200, 128
N = B * L
NC, NS = 2, 16
NW = NC * NS
PER_W = N // NW
CH = 128
SCH = 256
GPC = SCH // CH
NCHUNK = PER_W // SCH
NBUF = 3


def _pe_add_body(x_hbm, idx_hbm, pe_hbm, out_hbm,
                 idx_v, xb_v, sem_x, sem_g, sem_o):
    wid = lax.axis_index("s") * NC + lax.axis_index("c")
    base = wid * PER_W

    pltpu.sync_copy(idx_hbm.at[pl.ds(base, PER_W)], idx_v)

    def off(c):
        return base + c * SCH

    def fire_x(c, b):
        pltpu.async_copy(x_hbm.at[pl.ds(off(c), SCH)], xb_v.at[b], sem_x.at[b])

    def wait_x(c, b):
        pltpu.make_async_copy(
            x_hbm.at[pl.ds(off(c), SCH)], xb_v.at[b], sem_x.at[b]).wait()

    def fire_ga(c, b):
        for g in range(GPC):
            pltpu.async_copy(
                pe_hbm.at[idx_v.at[pl.ds(c * SCH + g * CH, CH)]],
                xb_v.at[b, pl.ds(g * CH, CH)], sem_g.at[b], add=True)

    def wait_ga(c, b):
        for g in range(GPC):
            pltpu.make_async_copy(
                pe_hbm.at[idx_v.at[pl.ds(c * SCH + g * CH, CH)]],
                xb_v.at[b, pl.ds(g * CH, CH)], sem_g.at[b]).wait()

    def fire_out(c, b):
        pltpu.async_copy(xb_v.at[b], out_hbm.at[pl.ds(off(c), SCH)],
                         sem_o.at[b])

    def wait_out(c, b):
        pltpu.make_async_copy(
            xb_v.at[b], out_hbm.at[pl.ds(off(c), SCH)], sem_o.at[b]).wait()

    for t in range(NCHUNK + 2):
        if t >= 2:
            c = t - 2
            wait_ga(c, c % NBUF)
            fire_out(c, c % NBUF)
        if 1 <= t <= NCHUNK:
            c = t - 1
            wait_x(c, c % NBUF)
            fire_ga(c, c % NBUF)
        if t < NCHUNK:
            if t >= NBUF:
                wait_out(t - NBUF, t % NBUF)
            fire_x(t, t % NBUF)
    for k in range(NBUF):
        c = NCHUNK - NBUF + k
        wait_out(c, c % NBUF)


@jax.jit
def kernel(x, tss_indexes, pe):
    xf = x.reshape(N, D)
    idx = tss_indexes.reshape(N).astype(jnp.int32)
    mesh = plsc.VectorSubcoreMesh(
        core_axis_name="c", subcore_axis_name="s",
        num_cores=NC, num_subcores=NS,
    )
    out = pl.kernel(
        _pe_add_body,
        out_type=jax.ShapeDtypeStruct((N, D), jnp.float32),
        mesh=mesh,
        scratch_types=[
            pltpu.VMEM((PER_W,), jnp.int32),
            pltpu.VMEM((NBUF, SCH, D), jnp.float32),
            pltpu.SemaphoreType.DMA((NBUF,)),
            pltpu.SemaphoreType.DMA((NBUF,)),
            pltpu.SemaphoreType.DMA((NBUF,)),
        ],
    )(xf, idx, pe)
    return out.reshape(B, L, D)
